# Initial kernel scaffold; baseline (speedup 1.0000x reference)
import functools

import jax
import jax.numpy as jnp
from jax import lax
from jax.experimental import pallas as pl
from jax.experimental.pallas import tpu as pltpu
from jax.experimental.pallas import tpu_sc as plsc

N = 10000
D_IN = 128
H = 64
E = 320000
NC, NS = 2, 16
NW = NC * NS
SUB = 128
KSUB = 10
CH = SUB * KSUB
NCHUNK = E // CH
TMAX = -(-NCHUNK // NW)
NPAD = 10016
ZROWS = NPAD // NS
CW = 16

_sc_mesh = plsc.VectorSubcoreMesh(core_axis_name="c", subcore_axis_name="s",
                                  num_cores=NC, num_subcores=NS)


@functools.partial(
    pl.kernel,
    out_type=jax.ShapeDtypeStruct((NC, NPAD, H), jnp.float32),
    mesh=_sc_mesh,
    scratch_types=[
        pltpu.VMEM((KSUB, SUB), jnp.int32),
        pltpu.VMEM((KSUB, SUB), jnp.int32),
        pltpu.VMEM((CH, H), jnp.float32),
        pltpu.VMEM_SHARED((NPAD, H), jnp.float32),
        pltpu.SemaphoreType.DMA,
    ],
)
def _sc_segsum(table_hbm, src_hbm, dst_hbm, zeros_hbm, out_hbm,
               idx_src, idx_dst, rows, acc, sem):
    c = lax.axis_index("c")
    s = lax.axis_index("s")
    w = s * NC + c
    pltpu.sync_copy(zeros_hbm, acc.at[pl.ds(s * ZROWS, ZROWS)])
    plsc.subcore_barrier()
    for t in range(TMAX):
        g = w + NW * t

        @pl.when(g < NCHUNK)
        def _():
            pltpu.sync_copy(src_hbm.at[g], idx_src)
            pltpu.sync_copy(dst_hbm.at[g], idx_dst)
            descs = [
                pltpu.async_copy(table_hbm.at[idx_src.at[j]],
                                 rows.at[pl.ds(j * SUB, SUB)], sem)
                for j in range(KSUB)
            ]
            for d in descs:
                d.wait()
            for j in range(KSUB):
                pltpu.sync_copy(rows.at[pl.ds(j * SUB, SUB)],
                                acc.at[idx_dst.at[j]], add=True)

    plsc.subcore_barrier()
    pltpu.sync_copy(acc.at[pl.ds(s * ZROWS, ZROWS)],
                    out_hbm.at[c, pl.ds(s * ZROWS, ZROWS)])


@functools.partial(
    pl.kernel,
    out_type=jax.ShapeDtypeStruct((NC, NPAD, CW), jnp.float32),
    mesh=_sc_mesh,
    scratch_types=[
        pltpu.VMEM((KSUB, SUB), jnp.int32),
        pltpu.VMEM((CH, CW), jnp.float32),
        pltpu.VMEM_SHARED((NPAD, CW), jnp.float32),
    ],
)
def _sc_counts(dst_hbm, ones_hbm, zeros_hbm, out_hbm, idx_dst, ones_v, acc):
    c = lax.axis_index("c")
    s = lax.axis_index("s")
    w = s * NC + c
    pltpu.sync_copy(zeros_hbm, acc.at[pl.ds(s * ZROWS, ZROWS)])
    pltpu.sync_copy(ones_hbm, ones_v)
    plsc.subcore_barrier()
    for t in range(TMAX):
        g = w + NW * t

        @pl.when(g < NCHUNK)
        def _():
            pltpu.sync_copy(dst_hbm.at[g], idx_dst)
            for j in range(KSUB):
                pltpu.sync_copy(ones_v.at[pl.ds(j * SUB, SUB)],
                                acc.at[idx_dst.at[j]], add=True)

    plsc.subcore_barrier()
    pltpu.sync_copy(acc.at[pl.ds(s * ZROWS, ZROWS)],
                    out_hbm.at[c, pl.ds(s * ZROWS, ZROWS)])


R = 1000


def _proj_body(x_ref, wt_ref, b_ref, o_ref):
    y = jnp.dot(x_ref[...], wt_ref[...], preferred_element_type=jnp.float32)
    o_ref[...] = jnp.maximum(y + b_ref[...], 0.0)


def _proj(x, wt, b):
    return pl.pallas_call(
        _proj_body,
        grid=(N // R,),
        in_specs=[
            pl.BlockSpec((R, D_IN), lambda i: (i, 0)),
            pl.BlockSpec((D_IN, H), lambda i: (0, 0)),
            pl.BlockSpec((1, H), lambda i: (0, 0)),
        ],
        out_specs=pl.BlockSpec((R, H), lambda i: (i, 0)),
        out_shape=jax.ShapeDtypeStruct((N, H), jnp.float32),
    )(x, wt, b)


def _combine_body(s0_ref, s1_ref, c0_ref, c1_ref, h_ref, wlt_ref, wrt_ref,
                  b_ref, o_ref):
    cnt = c0_ref[0][:, :1] + c1_ref[0][:, :1]
    inv = 1.0 / jnp.maximum(cnt, 1.0)
    agg = (s0_ref[0] + s1_ref[0]) * inv
    y = (jnp.dot(agg, wlt_ref[...], preferred_element_type=jnp.float32)
         + jnp.dot(h_ref[...], wrt_ref[...], preferred_element_type=jnp.float32)
         + b_ref[...])
    o_ref[...] = jnp.maximum(y, 0.0)


def _combine(parts, cnts, h, wlt, wrt, b):
    return pl.pallas_call(
        _combine_body,
        grid=(N // R,),
        in_specs=[
            pl.BlockSpec((1, R, H), lambda i: (0, i, 0)),
            pl.BlockSpec((1, R, H), lambda i: (1, i, 0)),
            pl.BlockSpec((1, R, CW), lambda i: (0, i, 0)),
            pl.BlockSpec((1, R, CW), lambda i: (1, i, 0)),
            pl.BlockSpec((R, H), lambda i: (i, 0)),
            pl.BlockSpec((H, H), lambda i: (0, 0)),
            pl.BlockSpec((H, H), lambda i: (0, 0)),
            pl.BlockSpec((1, H), lambda i: (0, 0)),
        ],
        out_specs=pl.BlockSpec((R, H), lambda i: (i, 0)),
        out_shape=jax.ShapeDtypeStruct((N, H), jnp.float32),
    )(parts, parts, cnts, cnts, h, wlt, wrt, b)


def kernel(x_user, x_item, edge_index_ui, edge_index_iu,
           W_in_user, b_in_user, W_in_item, b_in_item,
           Wl0_ui, bl0_ui, Wr0_ui, Wl0_iu, bl0_iu, Wr0_iu,
           Wl1_ui, bl1_ui, Wr1_ui, Wl1_iu, bl1_iu, Wr1_iu):
    src_ui = edge_index_ui[0].astype(jnp.int32).reshape(NCHUNK, KSUB, SUB)
    dst_ui = edge_index_ui[1].astype(jnp.int32).reshape(NCHUNK, KSUB, SUB)
    src_iu = edge_index_iu[0].astype(jnp.int32).reshape(NCHUNK, KSUB, SUB)
    dst_iu = edge_index_iu[1].astype(jnp.int32).reshape(NCHUNK, KSUB, SUB)

    zeros_h = jnp.zeros((ZROWS, H), jnp.float32)
    zeros_c = jnp.zeros((ZROWS, CW), jnp.float32)
    ones_c = jnp.ones((CH, CW), jnp.float32)

    h_u = _proj(x_user, W_in_user.T, b_in_user[None])
    h_i = _proj(x_item, W_in_item.T, b_in_item[None])

    cnt_i = _sc_counts(dst_ui, ones_c, zeros_c)
    cnt_u = _sc_counts(dst_iu, ones_c, zeros_c)

    for Wl_ui, bl_ui, Wr_ui, Wl_iu, bl_iu, Wr_iu in (
            (Wl0_ui, bl0_ui, Wr0_ui, Wl0_iu, bl0_iu, Wr0_iu),
            (Wl1_ui, bl1_ui, Wr1_ui, Wl1_iu, bl1_iu, Wr1_iu)):
        s_i = _sc_segsum(h_u, src_ui, dst_ui, zeros_h)
        s_u = _sc_segsum(h_i, src_iu, dst_iu, zeros_h)
        h_i_new = _combine(s_i, cnt_i, h_i, Wl_ui.T, Wr_ui.T, bl_ui[None])
        h_u_new = _combine(s_u, cnt_u, h_u, Wl_iu.T, Wr_iu.T, bl_iu[None])
        h_u, h_i = h_i_new if False else h_u_new, h_i_new

    return (h_u, h_i)


# speedup vs baseline: 3.1697x; 3.1697x over previous
import functools

import jax
import jax.numpy as jnp
from jax import lax
from jax.experimental import pallas as pl
from jax.experimental.pallas import tpu as pltpu
from jax.experimental.pallas import tpu_sc as plsc

N = 10000
D_IN = 128
H = 64
HP = 128
E = 320000
NC, NS = 2, 16
SUB = 128
KSUB = 5
CH = SUB * KSUB
NCHUNK = E // CH
TMAX = -(-NCHUNK // NS)
HALF = 5120
TRASH = HALF
APAD = 5248
AZR = APAD // NS
P = 2 * HALF
R = 512
GRID = P // R

_sc_mesh = plsc.VectorSubcoreMesh(core_axis_name="c", subcore_axis_name="s",
                                  num_cores=NC, num_subcores=NS)


@functools.partial(
    pl.kernel,
    out_type=jax.ShapeDtypeStruct((NC, APAD, HP), jnp.float32),
    mesh=_sc_mesh,
    scratch_types=[
        pltpu.VMEM((KSUB, SUB), jnp.int32),
        pltpu.VMEM((KSUB, SUB), jnp.int32),
        pltpu.VMEM((CH, HP), jnp.float32),
        pltpu.VMEM_SHARED((APAD, HP), jnp.float32),
        pltpu.SemaphoreType.DMA,
    ],
)
def _sc_segsum(table_hbm, src_hbm, dst_hbm, zeros_hbm, out_hbm,
               idx_src, idx_dst, rows, acc, sem):
    c = lax.axis_index("c")
    s = lax.axis_index("s")
    base = c * HALF
    pltpu.sync_copy(zeros_hbm, acc.at[pl.ds(s * AZR, AZR)])
    plsc.subcore_barrier()

    def round_body(t, _):
        g = s + NS * t

        @pl.when(g < NCHUNK)
        def _():
            pltpu.sync_copy(src_hbm.at[g], idx_src)
            pltpu.sync_copy(dst_hbm.at[g], idx_dst)
            descs = [
                pltpu.async_copy(table_hbm.at[idx_src.at[j]],
                                 rows.at[pl.ds(j * SUB, SUB)], sem)
                for j in range(KSUB)
            ]
            for j in range(KSUB):
                for k in range(SUB // 16):
                    v = idx_dst[j, pl.ds(k * 16, 16)] - base
                    ok = (v >= 0) & (v < HALF)
                    idx_dst[j, pl.ds(k * 16, 16)] = jnp.where(ok, v, TRASH)
            for d in descs:
                d.wait()
            for j in range(KSUB):
                pltpu.sync_copy(rows.at[pl.ds(j * SUB, SUB)],
                                acc.at[idx_dst.at[j]], add=True)

        return _

    lax.fori_loop(0, TMAX, round_body, None)
    plsc.subcore_barrier()
    pltpu.sync_copy(acc.at[pl.ds(s * AZR, AZR)],
                    out_hbm.at[c, pl.ds(s * AZR, AZR)])


def _pad_cols(y):
    rows = y.shape[0]
    return jnp.concatenate(
        [y, jnp.ones((rows, 1), jnp.float32),
         jnp.zeros((rows, HP - H - 1), jnp.float32)], axis=1)


def _proj_body(x_ref, wt_ref, b_ref, o_ref):
    y = jnp.dot(x_ref[...], wt_ref[...], preferred_element_type=jnp.float32)
    o_ref[...] = _pad_cols(jnp.maximum(y + b_ref[...], 0.0))


def _proj(x, wt, b):
    return pl.pallas_call(
        _proj_body,
        grid=(GRID,),
        in_specs=[
            pl.BlockSpec((R, D_IN), lambda i: (i, 0)),
            pl.BlockSpec((D_IN, H), lambda i: (0, 0)),
            pl.BlockSpec((1, H), lambda i: (0, 0)),
        ],
        out_specs=pl.BlockSpec((R, HP), lambda i: (i, 0)),
        out_shape=jax.ShapeDtypeStruct((P, HP), jnp.float32),
    )(x, wt, b)


def _combine_body(s_ref, h_ref, wlt_ref, wrt_ref, b_ref, o_ref):
    s = s_ref[0]
    inv = 1.0 / jnp.maximum(s[:, H:H + 1], 1.0)
    agg = s * inv
    y = (jnp.dot(agg, wlt_ref[...], preferred_element_type=jnp.float32)
         + jnp.dot(h_ref[...], wrt_ref[...], preferred_element_type=jnp.float32)
         + b_ref[...])
    o_ref[...] = _pad_cols(jnp.maximum(y, 0.0))


def _combine(parts, h, wlt_pad, wrt_pad, b):
    blocks_per_core = HALF // R
    return pl.pallas_call(
        _combine_body,
        grid=(GRID,),
        in_specs=[
            pl.BlockSpec((1, R, HP),
                         lambda i: (i // blocks_per_core,
                                    i % blocks_per_core, 0)),
            pl.BlockSpec((R, HP), lambda i: (i, 0)),
            pl.BlockSpec((HP, H), lambda i: (0, 0)),
            pl.BlockSpec((HP, H), lambda i: (0, 0)),
            pl.BlockSpec((1, H), lambda i: (0, 0)),
        ],
        out_specs=pl.BlockSpec((R, HP), lambda i: (i, 0)),
        out_shape=jax.ShapeDtypeStruct((P, HP), jnp.float32),
    )(parts, h, wlt_pad, wrt_pad, b)


def _pad_rows(wt):
    return jnp.concatenate([wt, jnp.zeros((HP - H, H), jnp.float32)], axis=0)


def kernel(x_user, x_item, edge_index_ui, edge_index_iu,
           W_in_user, b_in_user, W_in_item, b_in_item,
           Wl0_ui, bl0_ui, Wr0_ui, Wl0_iu, bl0_iu, Wr0_iu,
           Wl1_ui, bl1_ui, Wr1_ui, Wl1_iu, bl1_iu, Wr1_iu):
    src_ui = edge_index_ui[0].astype(jnp.int32).reshape(NCHUNK, KSUB, SUB)
    dst_ui = edge_index_ui[1].astype(jnp.int32).reshape(NCHUNK, KSUB, SUB)
    src_iu = edge_index_iu[0].astype(jnp.int32).reshape(NCHUNK, KSUB, SUB)
    dst_iu = edge_index_iu[1].astype(jnp.int32).reshape(NCHUNK, KSUB, SUB)

    zeros_h = jnp.zeros((AZR, HP), jnp.float32)

    h_u = _proj(x_user, W_in_user.T, b_in_user[None])
    h_i = _proj(x_item, W_in_item.T, b_in_item[None])

    for Wl_ui, bl_ui, Wr_ui, Wl_iu, bl_iu, Wr_iu in (
            (Wl0_ui, bl0_ui, Wr0_ui, Wl0_iu, bl0_iu, Wr0_iu),
            (Wl1_ui, bl1_ui, Wr1_ui, Wl1_iu, bl1_iu, Wr1_iu)):
        s_i = _sc_segsum(h_u, src_ui, dst_ui, zeros_h)
        s_u = _sc_segsum(h_i, src_iu, dst_iu, zeros_h)
        h_i_new = _combine(s_i, h_i, _pad_rows(Wl_ui.T), _pad_rows(Wr_ui.T),
                           bl_ui[None])
        h_u_new = _combine(s_u, h_u, _pad_rows(Wl_iu.T), _pad_rows(Wr_iu.T),
                           bl_iu[None])
        h_u, h_i = h_u_new, h_i_new

    return (h_u[:N, :H], h_i[:N, :H])


# speedup vs baseline: 3.2850x; 1.0364x over previous
import functools

import jax
import jax.numpy as jnp
from jax import lax
from jax.experimental import pallas as pl
from jax.experimental.pallas import tpu as pltpu
from jax.experimental.pallas import tpu_sc as plsc

N = 10000
D_IN = 128
H = 64
HP = 128
E = 320000
NC, NS = 2, 16
SUB = 128
KSUB = 5
CH = SUB * KSUB
NCHUNK = E // CH
TMAX = -(-NCHUNK // NS)
HALF = 5120
TRASH = HALF
APAD = 5248
AZR = APAD // NS
P = 2 * HALF
R = 512
GRID = P // R

_sc_mesh = plsc.VectorSubcoreMesh(core_axis_name="c", subcore_axis_name="s",
                                  num_cores=NC, num_subcores=NS)


@functools.partial(
    pl.kernel,
    out_type=jax.ShapeDtypeStruct((NC, APAD, HP), jnp.float32),
    mesh=_sc_mesh,
    scratch_types=[
        pltpu.VMEM((KSUB, SUB), jnp.int32),
        pltpu.VMEM((KSUB, SUB), jnp.int32),
        pltpu.VMEM((CH, HP), jnp.float32),
        pltpu.VMEM_SHARED((APAD, HP), jnp.float32),
        pltpu.SemaphoreType.DMA,
        pltpu.SemaphoreType.DMA,
        pltpu.SemaphoreType.DMA,
        pltpu.SemaphoreType.DMA,
        pltpu.SemaphoreType.DMA,
        pltpu.SemaphoreType.DMA,
    ],
)
def _sc_segsum(table_hbm, src_hbm, dst_hbm, zeros_hbm, out_hbm,
               idx_src, idx_dst, rows, acc, g0, g1, g2, g3, g4, ss):
    c = lax.axis_index("c")
    s = lax.axis_index("s")
    base = c * HALF
    gsems = (g0, g1, g2, g3, g4)
    pltpu.sync_copy(zeros_hbm, acc.at[pl.ds(s * AZR, AZR)])
    plsc.subcore_barrier()

    def round_body(t, carry):
        g = s + NS * t

        @pl.when(g < NCHUNK)
        def _():
            pltpu.sync_copy(src_hbm.at[g], idx_src)
            pltpu.sync_copy(dst_hbm.at[g], idx_dst)
            descs = [
                pltpu.async_copy(table_hbm.at[idx_src.at[j]],
                                 rows.at[pl.ds(j * SUB, SUB)], gsems[j])
                for j in range(KSUB)
            ]
            for j in range(KSUB):
                for k in range(SUB // 16):
                    v = idx_dst[j, pl.ds(k * 16, 16)] - base
                    ok = (v >= 0) & (v < HALF)
                    idx_dst[j, pl.ds(k * 16, 16)] = jnp.where(ok, v, TRASH)
            sdescs = []
            for j in range(KSUB):
                descs[j].wait()
                sdescs.append(
                    pltpu.async_copy(rows.at[pl.ds(j * SUB, SUB)],
                                     acc.at[idx_dst.at[j]], ss, add=True))
            for d in sdescs:
                d.wait()

        return carry

    lax.fori_loop(0, TMAX, round_body, None)
    plsc.subcore_barrier()
    pltpu.sync_copy(acc.at[pl.ds(s * AZR, AZR)],
                    out_hbm.at[c, pl.ds(s * AZR, AZR)])


def _pad_cols(y):
    rows = y.shape[0]
    return jnp.concatenate(
        [y, jnp.ones((rows, 1), jnp.float32),
         jnp.zeros((rows, HP - H - 1), jnp.float32)], axis=1)


def _proj_body(x_ref, wt_ref, b_ref, o_ref):
    y = jnp.dot(x_ref[...], wt_ref[...], preferred_element_type=jnp.float32)
    o_ref[...] = _pad_cols(jnp.maximum(y + b_ref[...], 0.0))


def _proj(x, wt, b):
    return pl.pallas_call(
        _proj_body,
        grid=(GRID,),
        in_specs=[
            pl.BlockSpec((R, D_IN), lambda i: (i, 0)),
            pl.BlockSpec((D_IN, H), lambda i: (0, 0)),
            pl.BlockSpec((1, H), lambda i: (0, 0)),
        ],
        out_specs=pl.BlockSpec((R, HP), lambda i: (i, 0)),
        out_shape=jax.ShapeDtypeStruct((P, HP), jnp.float32),
    )(x, wt, b)


def _combine_body(s_ref, h_ref, wlt_ref, wrt_ref, b_ref, o_ref):
    s = s_ref[0]
    inv = 1.0 / jnp.maximum(s[:, H:H + 1], 1.0)
    agg = s * inv
    y = (jnp.dot(agg, wlt_ref[...], preferred_element_type=jnp.float32)
         + jnp.dot(h_ref[...], wrt_ref[...], preferred_element_type=jnp.float32)
         + b_ref[...])
    o_ref[...] = _pad_cols(jnp.maximum(y, 0.0))


def _combine(parts, h, wlt_pad, wrt_pad, b):
    blocks_per_core = HALF // R
    return pl.pallas_call(
        _combine_body,
        grid=(GRID,),
        in_specs=[
            pl.BlockSpec((1, R, HP),
                         lambda i: (i // blocks_per_core,
                                    i % blocks_per_core, 0)),
            pl.BlockSpec((R, HP), lambda i: (i, 0)),
            pl.BlockSpec((HP, H), lambda i: (0, 0)),
            pl.BlockSpec((HP, H), lambda i: (0, 0)),
            pl.BlockSpec((1, H), lambda i: (0, 0)),
        ],
        out_specs=pl.BlockSpec((R, HP), lambda i: (i, 0)),
        out_shape=jax.ShapeDtypeStruct((P, HP), jnp.float32),
    )(parts, h, wlt_pad, wrt_pad, b)


def _pad_rows(wt):
    return jnp.concatenate([wt, jnp.zeros((HP - H, H), jnp.float32)], axis=0)


def kernel(x_user, x_item, edge_index_ui, edge_index_iu,
           W_in_user, b_in_user, W_in_item, b_in_item,
           Wl0_ui, bl0_ui, Wr0_ui, Wl0_iu, bl0_iu, Wr0_iu,
           Wl1_ui, bl1_ui, Wr1_ui, Wl1_iu, bl1_iu, Wr1_iu):
    src_ui = edge_index_ui[0].astype(jnp.int32).reshape(NCHUNK, KSUB, SUB)
    dst_ui = edge_index_ui[1].astype(jnp.int32).reshape(NCHUNK, KSUB, SUB)
    src_iu = edge_index_iu[0].astype(jnp.int32).reshape(NCHUNK, KSUB, SUB)
    dst_iu = edge_index_iu[1].astype(jnp.int32).reshape(NCHUNK, KSUB, SUB)

    zeros_h = jnp.zeros((AZR, HP), jnp.float32)

    h_u = _proj(x_user, W_in_user.T, b_in_user[None])
    h_i = _proj(x_item, W_in_item.T, b_in_item[None])

    for Wl_ui, bl_ui, Wr_ui, Wl_iu, bl_iu, Wr_iu in (
            (Wl0_ui, bl0_ui, Wr0_ui, Wl0_iu, bl0_iu, Wr0_iu),
            (Wl1_ui, bl1_ui, Wr1_ui, Wl1_iu, bl1_iu, Wr1_iu)):
        s_i = _sc_segsum(h_u, src_ui, dst_ui, zeros_h)
        s_u = _sc_segsum(h_i, src_iu, dst_iu, zeros_h)
        h_i_new = _combine(s_i, h_i, _pad_rows(Wl_ui.T), _pad_rows(Wr_ui.T),
                           bl_ui[None])
        h_u_new = _combine(s_u, h_u, _pad_rows(Wl_iu.T), _pad_rows(Wr_iu.T),
                           bl_iu[None])
        h_u, h_i = h_u_new, h_i_new

    return (h_u[:N, :H], h_i[:N, :H])
